# Initial kernel scaffold; baseline (speedup 1.0000x reference)
import jax
import jax.numpy as jnp
from jax.experimental import pallas as pl


def kernel(x, linear_w, linear_bias, ffm_tables):
    raise NotImplementedError("write your pallas kernel here")


# speedup vs baseline: 10.3417x; 10.3417x over previous
import functools

import jax
import jax.numpy as jnp
import numpy as np
from jax import lax
from jax.experimental import pallas as pl
from jax.experimental.pallas import tpu as pltpu
from jax.experimental.pallas import tpu_sc as plsc

F = 26
FS = 4000
V = F * FS
D = 16
B = 4096
L = 16
NC, NS = 2, 16
NW = NC * NS
BW = B // NW
P = F * (F - 1) // 2
NIDX = 2 * P
KCH = (NIDX + L - 1) // L
NPAD = KCH * L
XPAD = 32

_i, _j = np.triu_indices(F, k=1)
_s = np.zeros(NPAD, np.int32)
_t = np.zeros(NPAD, np.int32)
_s[0:NIDX:2] = _i
_t[0:NIDX:2] = _j
_s[1:NIDX:2] = _j
_t[1:NIDX:2] = _i
_S_ARR = np.ascontiguousarray(_s.reshape(KCH, L))
_TOFF_ARR = np.ascontiguousarray((_t * V).reshape(KCH, L))


def _ffm_body(xi_hbm, lw_hbm, tbl_hbm, s_hbm, t_hbm, out_hbm,
              xiv, sv, tv, idxv, rows, linv, outv, sem, sem2):
    wid = lax.axis_index("s") * NC + lax.axis_index("c")
    base = wid * BW
    pltpu.sync_copy(xi_hbm.at[pl.ds(base * XPAD, BW * XPAD)], xiv)
    pltpu.sync_copy(s_hbm, sv)
    pltpu.sync_copy(t_hbm, tv)
    pltpu.async_copy(lw_hbm.at[xiv], linv, sem2).wait()

    lane = lax.broadcasted_iota(jnp.int32, (L,), 0)
    lane0 = lane == 0

    def bstep(b, _):
        bvec = jnp.zeros((L,), jnp.int32) + b * XPAD
        for k in range(KCH):
            xis = plsc.load_gather(xiv, [bvec + sv[k]])
            idxv[pl.ds(k * L, L)] = xis + tv[k]
        pltpu.async_copy(tbl_hbm.at[idxv], rows, sem).wait()
        acc = linv[pl.ds(b * XPAD, L)] + linv[pl.ds(b * XPAD + L, L)]

        def pstep(p, a):
            return a + rows[2 * p] * rows[2 * p + 1]

        acc = lax.fori_loop(0, P, pstep, acc, unroll=8)
        tot = jnp.zeros((L,), jnp.float32) + jnp.sum(acc)
        plsc.store_scatter(outv, [jnp.zeros((L,), jnp.int32) + b], tot,
                           mask=lane0)
        return 0

    lax.fori_loop(0, BW, bstep, 0)
    for k in range(BW // L):
        z = outv[pl.ds(k * L, L)]
        outv[pl.ds(k * L, L)] = 1.0 / (1.0 + jnp.exp(-z))
    pltpu.sync_copy(outv, out_hbm.at[pl.ds(base, BW)])


@functools.partial(jax.jit, static_argnames=())
def _ffm_call(xi_pad, lw_aug, tbl_flat, s_arr, t_arr):
    mesh = plsc.VectorSubcoreMesh(
        core_axis_name="c", subcore_axis_name="s",
        num_cores=NC, num_subcores=NS)
    fn = pl.kernel(
        _ffm_body,
        out_type=jax.ShapeDtypeStruct((B,), jnp.float32),
        mesh=mesh,
        compiler_params=pltpu.CompilerParams(
            needs_layout_passes=False, use_tc_tiling_on_sc=False),
        scratch_types=[
            pltpu.VMEM((BW * XPAD,), jnp.int32),
            pltpu.VMEM((KCH, L), jnp.int32),
            pltpu.VMEM((KCH, L), jnp.int32),
            pltpu.VMEM((NPAD,), jnp.int32),
            pltpu.VMEM((NPAD, D), jnp.float32),
            pltpu.VMEM((BW * XPAD,), jnp.float32),
            pltpu.VMEM((BW,), jnp.float32),
            pltpu.SemaphoreType.DMA,
            pltpu.SemaphoreType.DMA,
        ],
    )
    return fn(xi_pad, lw_aug, tbl_flat, s_arr, t_arr)


def kernel(x, linear_w, linear_bias, ffm_tables):
    x = x.astype(jnp.int32)
    offs = jnp.arange(F, dtype=jnp.int32) * FS
    xi = x + offs[None, :]
    xi_pad = jnp.pad(xi, ((0, 0), (0, XPAD - F)),
                     constant_values=V).reshape(-1)
    lw_aug = jnp.concatenate(
        [linear_w.reshape(-1),
         jnp.broadcast_to(linear_bias / float(XPAD - F), (8,))])
    tbl_flat = ffm_tables.reshape(F * V, D)
    return _ffm_call(xi_pad, lw_aug, tbl_flat,
                     jnp.asarray(_S_ARR), jnp.asarray(_TOFF_ARR))


# speedup vs baseline: 11.2040x; 1.0834x over previous
import functools

import jax
import jax.numpy as jnp
import numpy as np
from jax import lax
from jax.experimental import pallas as pl
from jax.experimental.pallas import tpu as pltpu
from jax.experimental.pallas import tpu_sc as plsc

F = 26
FS = 4000
V = F * FS
D = 16
B = 4096
L = 16
NC, NS = 2, 16
NW = NC * NS
BW = B // NW
P = F * (F - 1) // 2
NIDX = 2 * P
KCH = (NIDX + L - 1) // L
NPAD = KCH * L
XPAD = 32

_i, _j = np.triu_indices(F, k=1)
_s = np.zeros(NPAD, np.int32)
_t = np.zeros(NPAD, np.int32)
_s[0:NIDX:2] = _i
_t[0:NIDX:2] = _j
_s[1:NIDX:2] = _j
_t[1:NIDX:2] = _i
_S_ARR = np.ascontiguousarray(_s.reshape(KCH, L))
_TOFF_ARR = np.ascontiguousarray((_t * V).reshape(KCH, L))


def _ffm_body(xi_hbm, lw_hbm, tbl_hbm, s_hbm, t_hbm, out_hbm,
              xiv, sv, tv, idxv0, idxv1, rows0, rows1, linv, outv,
              sem0, sem1, sem2):
    wid = lax.axis_index("s") * NC + lax.axis_index("c")
    base = wid * BW
    pltpu.sync_copy(xi_hbm.at[pl.ds(base * XPAD, BW * XPAD)], xiv)
    pltpu.sync_copy(s_hbm, sv)
    pltpu.sync_copy(t_hbm, tv)
    pltpu.async_copy(lw_hbm.at[xiv], linv, sem2).wait()

    lane = lax.broadcasted_iota(jnp.int32, (L,), 0)
    lane0 = lane == 0

    def fetch(b, idxv, rows, sem):
        bvec = jnp.zeros((L,), jnp.int32) + b * XPAD
        for k in range(KCH):
            xis = plsc.load_gather(xiv, [bvec + sv[k]])
            idxv[pl.ds(k * L, L)] = xis + tv[k]
        return pltpu.async_copy(tbl_hbm.at[idxv], rows, sem)

    def compute(b, rows):
        acc = linv[pl.ds(b * XPAD, L)] + linv[pl.ds(b * XPAD + L, L)]

        def pstep(p, a):
            return a + rows[2 * p] * rows[2 * p + 1]

        acc = lax.fori_loop(0, P, pstep, acc, unroll=8)
        tot = jnp.zeros((L,), jnp.float32) + jnp.sum(acc)
        plsc.store_scatter(outv, [jnp.zeros((L,), jnp.int32) + b], tot,
                           mask=lane0)

    fetch(0, idxv0, rows0, sem0)

    def bstep(g, _):
        b0 = 2 * g
        fetch(b0 + 1, idxv1, rows1, sem1)
        pltpu.make_async_copy(tbl_hbm.at[idxv0], rows0, sem0).wait()
        compute(b0, rows0)
        fetch(b0 + 2, idxv0, rows0, sem0)
        pltpu.make_async_copy(tbl_hbm.at[idxv1], rows1, sem1).wait()
        compute(b0 + 1, rows1)
        return 0

    lax.fori_loop(0, BW // 2 - 1, bstep, 0)
    fetch(BW - 1, idxv1, rows1, sem1)
    pltpu.make_async_copy(tbl_hbm.at[idxv0], rows0, sem0).wait()
    compute(BW - 2, rows0)
    pltpu.make_async_copy(tbl_hbm.at[idxv1], rows1, sem1).wait()
    compute(BW - 1, rows1)
    for k in range(BW // L):
        z = outv[pl.ds(k * L, L)]
        outv[pl.ds(k * L, L)] = 1.0 / (1.0 + jnp.exp(-z))
    pltpu.sync_copy(outv, out_hbm.at[pl.ds(base, BW)])


@functools.partial(jax.jit, static_argnames=())
def _ffm_call(xi_pad, lw_aug, tbl_flat, s_arr, t_arr):
    mesh = plsc.VectorSubcoreMesh(
        core_axis_name="c", subcore_axis_name="s",
        num_cores=NC, num_subcores=NS)
    fn = pl.kernel(
        _ffm_body,
        out_type=jax.ShapeDtypeStruct((B,), jnp.float32),
        mesh=mesh,
        compiler_params=pltpu.CompilerParams(
            needs_layout_passes=False, use_tc_tiling_on_sc=False),
        scratch_types=[
            pltpu.VMEM((BW * XPAD,), jnp.int32),
            pltpu.VMEM((KCH, L), jnp.int32),
            pltpu.VMEM((KCH, L), jnp.int32),
            pltpu.VMEM((NPAD,), jnp.int32),
            pltpu.VMEM((NPAD,), jnp.int32),
            pltpu.VMEM((NPAD, D), jnp.float32),
            pltpu.VMEM((NPAD, D), jnp.float32),
            pltpu.VMEM((BW * XPAD,), jnp.float32),
            pltpu.VMEM((BW,), jnp.float32),
            pltpu.SemaphoreType.DMA,
            pltpu.SemaphoreType.DMA,
            pltpu.SemaphoreType.DMA,
        ],
    )
    return fn(xi_pad, lw_aug, tbl_flat, s_arr, t_arr)


def kernel(x, linear_w, linear_bias, ffm_tables):
    x = x.astype(jnp.int32)
    offs = jnp.arange(F, dtype=jnp.int32) * FS
    xi = x + offs[None, :]
    xi_pad = jnp.pad(xi, ((0, 0), (0, XPAD - F)),
                     constant_values=V).reshape(-1)
    lw_aug = jnp.concatenate(
        [linear_w.reshape(-1),
         jnp.broadcast_to(linear_bias / float(XPAD - F), (8,))])
    tbl_flat = ffm_tables.reshape(F * V, D)
    return _ffm_call(xi_pad, lw_aug, tbl_flat,
                     jnp.asarray(_S_ARR), jnp.asarray(_TOFF_ARR))


# speedup vs baseline: 11.3421x; 1.0123x over previous
import functools

import jax
import jax.numpy as jnp
import numpy as np
from jax import lax
from jax.experimental import pallas as pl
from jax.experimental.pallas import tpu as pltpu
from jax.experimental.pallas import tpu_sc as plsc

F = 26
FS = 4000
V = F * FS
D = 16
B = 4096
L = 16
NC, NS = 2, 16
NW = NC * NS
BW = B // NW
P = F * (F - 1) // 2
NIDX = 2 * P
KCH = (NIDX + L - 1) // L
NPAD = KCH * L
XPAD = 32
G = 4
NG = BW // G

_i, _j = np.triu_indices(F, k=1)
_s = np.zeros(NPAD, np.int32)
_t = np.zeros(NPAD, np.int32)
_s[0:NIDX:2] = _i
_t[0:NIDX:2] = _j
_s[1:NIDX:2] = _j
_t[1:NIDX:2] = _i
_S_ARR = np.ascontiguousarray(_s.reshape(KCH, L))
_TOFF_ARR = np.ascontiguousarray((_t * V).reshape(KCH, L))


def _ffm_body(xi_hbm, lw_hbm, tbl_hbm, s_hbm, t_hbm, out_hbm,
              xiv, sv, tv, idxv0, idxv1, rows0, rows1, linv, outv,
              sem0, sem1, sem2):
    wid = lax.axis_index("s") * NC + lax.axis_index("c")
    base = wid * BW
    pltpu.sync_copy(xi_hbm.at[pl.ds(base * XPAD, BW * XPAD)], xiv)
    pltpu.sync_copy(s_hbm, sv)
    pltpu.sync_copy(t_hbm, tv)
    lin_dma = pltpu.async_copy(lw_hbm.at[xiv], linv, sem2)

    lane = lax.broadcasted_iota(jnp.int32, (L,), 0)
    lane0 = lane == 0

    def fetch(g, idxv, rows, sem):
        for r in range(G):
            bvec = jnp.zeros((L,), jnp.int32) + (g * G + r) * XPAD
            for k in range(KCH):
                xis = plsc.load_gather(xiv, [bvec + sv[k]])
                idxv[pl.ds((r * KCH + k) * L, L)] = xis + tv[k]
        pltpu.async_copy(tbl_hbm.at[idxv], rows, sem)

    def compute(g, rows):
        for r in range(G):
            b = g * G + r
            acc = linv[pl.ds(b * XPAD, L)] + linv[pl.ds(b * XPAD + L, L)]

            def pstep(p, a, _r=r):
                return a + rows[_r * NPAD + 2 * p] * rows[_r * NPAD + 2 * p + 1]

            acc = lax.fori_loop(0, P, pstep, acc, unroll=8)
            tot = jnp.zeros((L,), jnp.float32) + jnp.sum(acc)
            plsc.store_scatter(outv, [jnp.zeros((L,), jnp.int32) + b], tot,
                               mask=lane0)

    fetch(0, idxv0, rows0, sem0)
    lin_dma.wait()

    def gstep(h, _):
        g0 = 2 * h
        fetch(g0 + 1, idxv1, rows1, sem1)
        pltpu.make_async_copy(tbl_hbm.at[idxv0], rows0, sem0).wait()
        compute(g0, rows0)
        fetch(g0 + 2, idxv0, rows0, sem0)
        pltpu.make_async_copy(tbl_hbm.at[idxv1], rows1, sem1).wait()
        compute(g0 + 1, rows1)
        return 0

    lax.fori_loop(0, NG // 2 - 1, gstep, 0)
    fetch(NG - 1, idxv1, rows1, sem1)
    pltpu.make_async_copy(tbl_hbm.at[idxv0], rows0, sem0).wait()
    compute(NG - 2, rows0)
    pltpu.make_async_copy(tbl_hbm.at[idxv1], rows1, sem1).wait()
    compute(NG - 1, rows1)
    for k in range(BW // L):
        z = outv[pl.ds(k * L, L)]
        outv[pl.ds(k * L, L)] = 1.0 / (1.0 + jnp.exp(-z))
    pltpu.sync_copy(outv, out_hbm.at[pl.ds(base, BW)])


@functools.partial(jax.jit, static_argnames=())
def _ffm_call(xi_pad, lw_aug, tbl_flat, s_arr, t_arr):
    mesh = plsc.VectorSubcoreMesh(
        core_axis_name="c", subcore_axis_name="s",
        num_cores=NC, num_subcores=NS)
    fn = pl.kernel(
        _ffm_body,
        out_type=jax.ShapeDtypeStruct((B,), jnp.float32),
        mesh=mesh,
        compiler_params=pltpu.CompilerParams(
            needs_layout_passes=False, use_tc_tiling_on_sc=False),
        scratch_types=[
            pltpu.VMEM((BW * XPAD,), jnp.int32),
            pltpu.VMEM((KCH, L), jnp.int32),
            pltpu.VMEM((KCH, L), jnp.int32),
            pltpu.VMEM((G * NPAD,), jnp.int32),
            pltpu.VMEM((G * NPAD,), jnp.int32),
            pltpu.VMEM((G * NPAD, D), jnp.float32),
            pltpu.VMEM((G * NPAD, D), jnp.float32),
            pltpu.VMEM((BW * XPAD,), jnp.float32),
            pltpu.VMEM((BW,), jnp.float32),
            pltpu.SemaphoreType.DMA,
            pltpu.SemaphoreType.DMA,
            pltpu.SemaphoreType.DMA,
        ],
    )
    return fn(xi_pad, lw_aug, tbl_flat, s_arr, t_arr)


def kernel(x, linear_w, linear_bias, ffm_tables):
    x = x.astype(jnp.int32)
    offs = jnp.arange(F, dtype=jnp.int32) * FS
    xi = x + offs[None, :]
    xi_pad = jnp.pad(xi, ((0, 0), (0, XPAD - F)),
                     constant_values=V).reshape(-1)
    lw_aug = jnp.concatenate(
        [linear_w.reshape(-1),
         jnp.broadcast_to(linear_bias / float(XPAD - F), (8,))])
    tbl_flat = ffm_tables.reshape(F * V, D)
    return _ffm_call(xi_pad, lw_aug, tbl_flat,
                     jnp.asarray(_S_ARR), jnp.asarray(_TOFF_ARR))


# speedup vs baseline: 44.2331x; 3.8999x over previous
import functools

import jax
import jax.numpy as jnp
import numpy as np
from jax import lax
from jax.experimental import pallas as pl
from jax.experimental.pallas import tpu as pltpu
from jax.experimental.pallas import tpu_sc as plsc

F = 26
FS = 4000
V = F * FS
D = 16
B = 4096
L = 16
NC, NS = 2, 16
NW = NC * NS
BW = B // NW
P = F * (F - 1) // 2
NR = (P + NW - 1) // NW
XPAD = 32
WIN = 4096
DH = 8
NGRP = B // L

_i, _j = np.triu_indices(F, k=1)
_prm = np.zeros((NR * NW, 16), np.int32)
for _p in range(P):
    fi, fj = int(_i[_p]), int(_j[_p])
    _prm[_p, 0] = fi * B
    _prm[_p, 1] = fj * B
    _prm[_p, 2] = (fi * FS) // 128 * 128
    _prm[_p, 3] = (fj * FS) // 128 * 128
    _prm[_p, 4] = (fi * FS) % 128
    _prm[_p, 5] = (fj * FS) % 128
    _prm[_p, 6] = fj
    _prm[_p, 7] = fi
_PRM = _prm.reshape(-1)


def _sweep_body(xtf_hbm, xiq_hbm, lw_hbm, tbl_hbm, prm_hbm, out_hbm,
                prmv, xa, xb, slA, slB, accv, xiv, linv,
                sem_a, sem_b, sem_l):
    wid = lax.axis_index("s") * NC + lax.axis_index("c")
    zero16 = jnp.zeros((L,), jnp.float32)

    def zstep(g, _):
        accv[pl.ds(g * L, L)] = zero16
        return 0

    lax.fori_loop(0, NGRP, zstep, 0)
    pltpu.sync_copy(prm_hbm, prmv)
    pltpu.sync_copy(xiq_hbm.at[pl.ds(wid * (XPAD * BW), XPAD * BW)], xiv)
    pltpu.async_copy(lw_hbm.at[xiv], linv, sem_l)

    def do_pair(p):
        row = prmv[pl.ds(p * 16, 16)]
        xoff_a = pl.multiple_of(row[0], 8)
        xoff_b = pl.multiple_of(row[1], 8)
        alA = pl.multiple_of(row[2], 128)
        alB = pl.multiple_of(row[3], 128)
        mA = row[4]
        mB = row[5]
        tA = row[6]
        tB = row[7]
        pltpu.sync_copy(xtf_hbm.at[pl.ds(xoff_a, B)], xa)
        pltpu.sync_copy(xtf_hbm.at[pl.ds(xoff_b, B)], xb)
        for dp in range(D // DH):
            pltpu.async_copy(
                tbl_hbm.at[tA, pl.ds(dp * DH, DH), pl.ds(alA, WIN)],
                slA, sem_a)
            pltpu.async_copy(
                tbl_hbm.at[tB, pl.ds(dp * DH, DH), pl.ds(alB, WIN)],
                slB, sem_b)
            pltpu.make_async_copy(
                tbl_hbm.at[tA, pl.ds(dp * DH, DH), pl.ds(alA, WIN)],
                slA, sem_a).wait()
            pltpu.make_async_copy(
                tbl_hbm.at[tB, pl.ds(dp * DH, DH), pl.ds(alB, WIN)],
                slB, sem_b).wait()

            def gstep(g, _):
                xav = xa[pl.ds(g * L, L)] + mA
                xbv = xb[pl.ds(g * L, L)] + mB
                acc = zero16
                for d in range(DH):
                    dvec = jnp.zeros((L,), jnp.int32) + d
                    va = plsc.load_gather(slA, [dvec, xav])
                    vb = plsc.load_gather(slB, [dvec, xbv])
                    acc = acc + va * vb
                accv[pl.ds(g * L, L)] = accv[pl.ds(g * L, L)] + acc
                return 0

            lax.fori_loop(0, NGRP, gstep, 0)

    for r in range(NR - 1):
        do_pair(r * NW + wid)
    p_last = (NR - 1) * NW + wid

    @pl.when(p_last < P)
    def _():
        do_pair(p_last)

    pltpu.make_async_copy(lw_hbm.at[xiv], linv, sem_l).wait()
    for g in range(BW // L):
        t = zero16
        for k in range(XPAD):
            t = t + linv[pl.ds(k * BW + g * L, L)]
        sl = pl.ds(wid * BW + g * L, L)
        accv[sl] = accv[sl] + t
    pltpu.sync_copy(accv, out_hbm.at[pl.ds(wid * B, B)])


def _comb_body(part_hbm, out_hbm, bufv, outv, sem):
    wid = lax.axis_index("s") * NC + lax.axis_index("c")
    for w2 in range(NW):
        pltpu.async_copy(part_hbm.at[pl.ds(w2 * B + wid * BW, BW)],
                         bufv.at[pl.ds(w2 * BW, BW)], sem)
    for w2 in range(NW):
        pltpu.make_async_copy(part_hbm.at[pl.ds(w2 * B + wid * BW, BW)],
                              bufv.at[pl.ds(w2 * BW, BW)], sem).wait()
    zero16 = jnp.zeros((L,), jnp.float32)
    for g in range(BW // L):
        t = zero16
        for w2 in range(NW):
            t = t + bufv[pl.ds(w2 * BW + g * L, L)]
        outv[pl.ds(g * L, L)] = 1.0 / (1.0 + jnp.exp(-t))
    pltpu.sync_copy(outv, out_hbm.at[pl.ds(wid * BW, BW)])


@functools.partial(jax.jit, static_argnames=())
def _ffm_call(xtf, xiq, lw_aug, tbl_t, prm):
    mesh = plsc.VectorSubcoreMesh(
        core_axis_name="c", subcore_axis_name="s",
        num_cores=NC, num_subcores=NS)
    cparams = pltpu.CompilerParams(
        needs_layout_passes=False, use_tc_tiling_on_sc=True,
        disable_bounds_checks=True)
    sweep = pl.kernel(
        _sweep_body,
        out_type=jax.ShapeDtypeStruct((NW * B,), jnp.float32),
        mesh=mesh,
        compiler_params=cparams,
        scratch_types=[
            pltpu.VMEM((NR * NW * 16,), jnp.int32),
            pltpu.VMEM((B,), jnp.int32),
            pltpu.VMEM((B,), jnp.int32),
            pltpu.VMEM((DH, WIN), jnp.float32),
            pltpu.VMEM((DH, WIN), jnp.float32),
            pltpu.VMEM((B,), jnp.float32),
            pltpu.VMEM((XPAD * BW,), jnp.int32),
            pltpu.VMEM((XPAD * BW,), jnp.float32),
            pltpu.SemaphoreType.DMA,
            pltpu.SemaphoreType.DMA,
            pltpu.SemaphoreType.DMA,
        ],
    )
    parts = sweep(xtf, xiq, lw_aug, tbl_t, prm)
    comb = pl.kernel(
        _comb_body,
        out_type=jax.ShapeDtypeStruct((B,), jnp.float32),
        mesh=mesh,
        compiler_params=cparams,
        scratch_types=[
            pltpu.VMEM((B,), jnp.float32),
            pltpu.VMEM((BW,), jnp.float32),
            pltpu.SemaphoreType.DMA,
        ],
    )
    return comb(parts)


def kernel(x, linear_w, linear_bias, ffm_tables):
    x = x.astype(jnp.int32)
    xtf = jnp.transpose(x).reshape(-1)
    offs = jnp.arange(F, dtype=jnp.int32) * FS
    xi = x + offs[None, :]
    xi_pad = jnp.pad(xi, ((0, 0), (0, XPAD - F)), constant_values=V)
    xiq = xi_pad.reshape(NW, BW, XPAD).transpose(0, 2, 1).reshape(-1)
    lw_aug = jnp.concatenate(
        [linear_w.reshape(-1),
         jnp.broadcast_to(linear_bias / float(XPAD - F), (8,))])
    tbl_t = jnp.transpose(ffm_tables, (0, 2, 1))
    return _ffm_call(xtf, xiq, lw_aug, tbl_t, jnp.asarray(_PRM))
